# Initial kernel scaffold; baseline (speedup 1.0000x reference)
import jax
import jax.numpy as jnp
from jax.experimental import pallas as pl


def kernel(data, params, edge_index, action_mask):
    raise NotImplementedError("write your pallas kernel here")


# speedup vs baseline: 4.8104x; 4.8104x over previous
import jax
import jax.numpy as jnp
from jax.experimental import pallas as pl
from jax.experimental.pallas import tpu as pltpu

N = 97
NP = 128
E = 1552 + N
EP = 1664
CAP = 64
REG = 64

_HIGH = jax.lax.Precision.HIGHEST
f32 = jnp.float32


def _onehots(src_ref, dst_ref):
    lane = jax.lax.broadcasted_iota(jnp.int32, (EP, NP), 1)
    s_oh = (src_ref[...] == lane).astype(f32)
    d_oh = (dst_ref[...] == lane).astype(f32)
    return s_oh, d_oh


def _ranks(d_oh):
    ri = jax.lax.broadcasted_iota(jnp.int32, (128, 128), 0)
    ci = jax.lax.broadcasted_iota(jnp.int32, (128, 128), 1)
    tri = jnp.where(ci < ri, 1.0, 0.0).astype(jnp.bfloat16)
    d_oh_b = d_oh.astype(jnp.bfloat16)
    blocks = []
    prefix = jnp.zeros((1, NP), f32)
    for i in range(EP // 128):
        dbi = d_oh_b[128 * i:128 * (i + 1), :]
        part = jax.lax.dot_general(tri, dbi, (((1,), (0,)), ((), ())),
                                   preferred_element_type=f32)
        blocks.append(part + prefix)
        prefix = prefix + jnp.sum(dbi.astype(f32), axis=0, keepdims=True)
    C = jnp.concatenate(blocks, axis=0)
    r = jnp.sum(C * d_oh, axis=1, keepdims=True)
    return jnp.minimum(r, float(CAP - 1)).astype(jnp.int32)


def _table_sum(v, nch_real, shift, d_oh, r_int):
    nch = 1 << shift
    if nch > nch_real:
        v = jnp.concatenate(
            [v, jnp.zeros((EP, nch - nch_real), f32)], axis=1)
    J = CAP * nch
    j_iota = jax.lax.broadcasted_iota(jnp.int32, (EP, J), 1)
    K_rep = (jax.lax.shift_right_logical(j_iota, shift) == r_int).astype(f32)
    c_row = jax.lax.broadcasted_iota(jnp.int32, (nch, J), 0)
    j_row = jax.lax.broadcasted_iota(jnp.int32, (nch, J), 1)
    S = ((j_row & (nch - 1)) == c_row).astype(f32)
    V_tile = jax.lax.dot_general(v, S, (((1,), (0,)), ((), ())),
                                 preferred_element_type=f32, precision=_HIGH)
    A_big = K_rep * V_tile
    T = jax.lax.dot_general(d_oh, A_big, (((0,), (0,)), ((), ())),
                            preferred_element_type=f32, precision=_HIGH)
    acc = T[:, 0:nch]
    for k in range(1, CAP):
        acc = acc + T[:, k * nch:(k + 1) * nch]
    return acc[:, 0:nch_real]


def _seq_colsum(p, H, C):
    cols = []
    for h in range(H):
        s = p[:, h * C:h * C + 1]
        for c in range(1, C):
            s = s + p[:, h * C + c:h * C + c + 1]
        cols.append(s)
    return cols[0] if H == 1 else jnp.concatenate(cols, axis=1)


def _gat_layer(s_oh, d_oh, r_int, x, w_ref, asv_ref, adv_ref, b_ref, H, C,
               hc_shift, h_shift, windowed):
    HC = H * C
    xh = jnp.dot(x, w_ref[...], preferred_element_type=f32)
    al_s = _seq_colsum(xh * asv_ref[...], H, C)
    al_d = _seq_colsum(xh * adv_ref[...], H, C)
    ii = jax.lax.broadcasted_iota(jnp.int32, (NP, NP), 0)
    jj = jax.lax.broadcasted_iota(jnp.int32, (NP, NP), 1)
    eye = (ii == jj).astype(f32)
    ltri = (jj < ii).astype(f32)
    alsT = jax.lax.dot_general(al_s, eye, (((0,), (0,)), ((), ())),
                               preferred_element_type=f32, precision=_HIGH)
    A_cnt = jax.lax.dot_general(d_oh.astype(jnp.bfloat16),
                                s_oh.astype(jnp.bfloat16),
                                (((0,), (0,)), ((), ())),
                                preferred_element_type=f32)
    has_edge = A_cnt > 0.0
    emax_cols = []
    for h in range(H):
        L = al_d[:, h:h + 1] + alsT[h:h + 1, :]
        L = jnp.where(L >= 0.0, L, 0.2 * L)
        M = jnp.where(has_edge, L, -1e30)
        emax_cols.append(jnp.max(M, axis=1, keepdims=True))
    emax = emax_cols[0] if H == 1 else jnp.concatenate(emax_cols, axis=1)
    als_e = jax.lax.dot_general(s_oh, al_s, (((1,), (0,)), ((), ())),
                                preferred_element_type=f32, precision=_HIGH)
    ald_e = jax.lax.dot_general(d_oh, al_d, (((1,), (0,)), ((), ())),
                                preferred_element_type=f32, precision=_HIGH)
    e_e = als_e + ald_e
    e_e = jnp.where(e_e >= 0.0, e_e, 0.2 * e_e)
    emax_e = jax.lax.dot_general(d_oh, emax, (((1,), (0,)), ((), ())),
                                 preferred_element_type=f32, precision=_HIGH)
    ee = jnp.exp(e_e - emax_e)
    if windowed:
        deg = jnp.sum(A_cnt, axis=1, keepdims=True)
        off = jax.lax.dot_general(ltri, deg, (((1,), (0,)), ((), ())),
                                  preferred_element_type=f32,
                                  precision=_HIGH)
        off_e = jax.lax.dot_general(d_oh, off, (((1,), (0,)), ((), ())),
                                    preferred_element_type=f32,
                                    precision=_HIGH)
        pos_e = off_e + r_int.astype(f32)
        in_first = (jnp.floor(pos_e / 96.0) ==
                    jnp.floor(off_e / 96.0)).astype(f32)
        denom = (_table_sum(ee * in_first, H, h_shift, d_oh, r_int) +
                 _table_sum(ee * (1.0 - in_first), H, h_shift, d_oh, r_int))
    else:
        denom = _table_sum(ee, H, h_shift, d_oh, r_int)
    den_e = jax.lax.dot_general(d_oh, denom, (((1,), (0,)), ((), ())),
                                preferred_element_type=f32, precision=_HIGH)
    alpha = ee / (den_e + 1e-16)
    xh_e = jax.lax.dot_general(s_oh, xh, (((1,), (0,)), ((), ())),
                               preferred_element_type=f32, precision=_HIGH)
    alpha_full = jnp.concatenate(
        [alpha[:, h:h + 1] for h in range(H) for _ in range(C)], axis=1)
    vals = xh_e * alpha_full
    out = _table_sum(vals, HC, hc_shift, d_oh, r_int)
    return out + b_ref[...]


def _k1(src_ref, dst_ref, x_ref, w_ref, asv_ref, adv_ref, b_ref,
        xout_ref, r_ref):
    s_oh, d_oh = _onehots(src_ref, dst_ref)
    r_int = _ranks(d_oh)
    r_ref[...] = r_int
    xout_ref[...] = _gat_layer(s_oh, d_oh, r_int, x_ref[...], w_ref,
                               asv_ref, adv_ref, b_ref, 2, 6, 4, 1, True)


def _k2(src_ref, dst_ref, r_ref, x_ref, w_ref, asv_ref, adv_ref, b_ref,
        xout_ref):
    s_oh, d_oh = _onehots(src_ref, dst_ref)
    xout_ref[...] = _gat_layer(s_oh, d_oh, r_ref[...], x_ref[...], w_ref,
                               asv_ref, adv_ref, b_ref, 2, 3, 3, 1, True)


def _k3(src_ref, dst_ref, r_ref, x_ref, w_ref, asv_ref, adv_ref, b_ref,
        nmask_ref, cw1_ref, cb1_ref, cw2_ref, cb2_ref,
        lw1_ref, lb1_ref, lw2_ref, lb2_ref, amask_ref, gum_ref,
        probs_ref, value_ref, action_ref):
    s_oh, d_oh = _onehots(src_ref, dst_ref)
    x3 = _gat_layer(s_oh, d_oh, r_ref[...], x_ref[...], w_ref,
                    asv_ref, adv_ref, b_ref, 1, 1, 0, 0, False)
    xf = x3 * nmask_ref[...]

    v1 = jax.lax.dot_general(xf, cw1_ref[...], (((0,), (0,)), ((), ())),
                             preferred_element_type=f32) + cb1_ref[...]
    value_ref[...] = jnp.dot(v1, cw2_ref[...],
                             preferred_element_type=f32) + cb2_ref[...]

    h1 = jax.lax.dot_general(xf, lw1_ref[...], (((0,), (0,)), ((), ())),
                             preferred_element_type=f32) + lb1_ref[...]
    h2 = jnp.dot(h1, lw2_ref[...], preferred_element_type=f32) + lb2_ref[...]
    logits = jnp.where(amask_ref[...] > 0.0, jnp.tanh(h2), -999999.0)
    lmax = jnp.max(logits, axis=1, keepdims=True)
    ex = jnp.exp(logits - lmax)
    probs = ex / jnp.sum(ex, axis=1, keepdims=True)
    probs_ref[...] = probs

    lg = jnp.log(probs + 1e-20) + gum_ref[...]
    gmax = jnp.max(lg, axis=1, keepdims=True)
    idx = jax.lax.broadcasted_iota(jnp.int32, (1, REG), 1)
    action_ref[...] = jnp.min(jnp.where(lg == gmax, idx, REG),
                              axis=1, keepdims=True)


_PARAMS = pltpu.CompilerParams(vmem_limit_bytes=100 * 1024 * 1024)


def kernel(data, params, edge_index, action_mask):
    p = params

    loop = jnp.arange(N, dtype=edge_index.dtype)
    src = jnp.concatenate([edge_index[0], loop])
    dst = jnp.concatenate([edge_index[1], loop])
    src = jnp.pad(src, (0, EP - E), constant_values=NP - 1).reshape(EP, 1)
    dst = jnp.pad(dst, (0, EP - E), constant_values=NP - 1).reshape(EP, 1)

    x0 = jnp.pad(data.astype(f32), ((0, NP - N), (0, 0)))

    dk = jax.random.key(42)
    keep1 = jnp.pad(jax.random.bernoulli(jax.random.fold_in(dk, 0), 0.5,
                                         (N, 12)), ((0, NP - N), (0, 0)))
    keep2 = jnp.pad(jax.random.bernoulli(jax.random.fold_in(dk, 1), 0.5,
                                         (N, 6)), ((0, NP - N), (0, 0)))
    gum = jax.random.gumbel(jax.random.key(7), (1, REG), f32)

    nmask = (jnp.arange(NP, dtype=jnp.int32) < N).astype(f32).reshape(NP, 1)

    def flat(a):
        return a.astype(f32).reshape(1, -1)

    x1_raw, r_int = pl.pallas_call(
        _k1,
        out_shape=(jax.ShapeDtypeStruct((NP, 12), f32),
                   jax.ShapeDtypeStruct((EP, 1), jnp.int32)),
        compiler_params=_PARAMS,
    )(src, dst, x0, p['W1'].astype(f32), flat(p['as1']), flat(p['ad1']),
      flat(p['b1']))

    x1 = jnp.where(keep1, jax.nn.elu(x1_raw) / (1.0 - 0.5), 0.0)

    x2_raw = pl.pallas_call(
        _k2,
        out_shape=jax.ShapeDtypeStruct((NP, 6), f32),
        compiler_params=_PARAMS,
    )(src, dst, r_int, x1, p['W2'].astype(f32), flat(p['as2']),
      flat(p['ad2']), flat(p['b2']))

    x2 = jnp.where(keep2, jax.nn.elu(x2_raw) / (1.0 - 0.5), 0.0)

    probs, value, action = pl.pallas_call(
        _k3,
        out_shape=(jax.ShapeDtypeStruct((1, REG), f32),
                   jax.ShapeDtypeStruct((1, 1), f32),
                   jax.ShapeDtypeStruct((1, 1), jnp.int32)),
        compiler_params=_PARAMS,
    )(src, dst, r_int, x2, p['W3'].astype(f32), flat(p['as3']),
      flat(p['ad3']), flat(p['b3']), nmask,
      jnp.pad(p['c_w1'].astype(f32), ((0, NP - N), (0, 0))), flat(p['c_b1']),
      p['c_w2'].astype(f32), flat(p['c_b2']),
      jnp.pad(p['l1_w1'].astype(f32), ((0, NP - N), (0, 0))), flat(p['l1_b1']),
      p['l1_w2'].astype(f32), flat(p['l1_b2']),
      action_mask.astype(f32).reshape(1, REG), gum)

    return (probs, value, action.reshape(1, 1), data)
